# baseline (speedup 1.0000x reference)
import functools

import jax
import jax.numpy as jnp
from jax import lax
from jax.experimental import pallas as pl
from jax.experimental.pallas import tpu as pltpu
from jax.experimental.pallas import tpu_sc as plsc

_NUM_PROMPTS = 12
_PROMPT_LEN = 20
_HIDDEN = 4096
_BATCH = 1024

_NC = 2
_NS = 16
_NW = _NC * _NS

_BPW = _BATCH // _NW
_IDX_PAD = 24
_PIECES = ((0, 8), (8, 8), (16, 4))


def _sc_gather(ridx, table2d):
    mesh = plsc.VectorSubcoreMesh(core_axis_name="c", subcore_axis_name="s")

    @functools.partial(
        pl.kernel,
        mesh=mesh,
        compiler_params=pltpu.CompilerParams(use_tc_tiling_on_sc=True),
        out_type=jax.ShapeDtypeStruct((_BATCH, _PROMPT_LEN, _HIDDEN),
                                      jnp.float32),
        scratch_types=(
            [pltpu.VMEM((_BPW, _IDX_PAD), jnp.int32)]
            + [pltpu.VMEM((ln, _HIDDEN), jnp.float32) for _, ln in _PIECES]
            + [pltpu.SemaphoreType.DMA] * 6
        ),
    )
    def k(ridx_hbm, table_hbm, out_hbm, ridx_v, buf0, buf1, buf2,
          g0, g1, g2, s0, s1, s2):
        bufs = (buf0, buf1, buf2)
        gsems = (g0, g1, g2)
        ssems = (s0, s1, s2)
        wid = lax.axis_index("s") * _NC + lax.axis_index("c")
        base = wid * _BPW

        pltpu.sync_copy(ridx_hbm.at[pl.ds(base, _BPW)], ridx_v)

        def body(j, carry):
            bb = base + j
            for s, (so, ln) in enumerate(_PIECES):
                @pl.when(j > 0)
                def _wait_prev_store():
                    pltpu.make_async_copy(
                        bufs[s],
                        out_hbm.at[bb - 1, pl.ds(so, ln), :],
                        ssems[s],
                    ).wait()

                pltpu.make_async_copy(
                    table_hbm.at[ridx_v.at[j, pl.ds(so, ln)]],
                    bufs[s],
                    gsems[s],
                ).start()
            for s, (so, ln) in enumerate(_PIECES):
                pltpu.make_async_copy(
                    table_hbm.at[ridx_v.at[j, pl.ds(so, ln)]],
                    bufs[s],
                    gsems[s],
                ).wait()
                pltpu.make_async_copy(
                    bufs[s],
                    out_hbm.at[bb, pl.ds(so, ln), :],
                    ssems[s],
                ).start()
            return carry

        lax.fori_loop(0, _BPW, body, 0)

        for s, (so, ln) in enumerate(_PIECES):
            pltpu.make_async_copy(
                bufs[s],
                out_hbm.at[base + _BPW - 1, pl.ds(so, ln), :],
                ssems[s],
            ).wait()

    return k(ridx, table2d)


def kernel(cultural_context, cultural_prompts):
    idx = cultural_context.astype(jnp.int32)
    t = jnp.minimum(jnp.arange(_IDX_PAD, dtype=jnp.int32), _PROMPT_LEN - 1)
    ridx = idx[:, None] * _PROMPT_LEN + t[None, :]
    table2d = cultural_prompts.reshape(_NUM_PROMPTS * _PROMPT_LEN, _HIDDEN)
    return _sc_gather(ridx, table2d)


# speedup vs baseline: 2.0189x; 2.0189x over previous
import functools

import jax
import jax.numpy as jnp
from jax import lax
from jax.experimental import pallas as pl
from jax.experimental.pallas import tpu as pltpu
from jax.experimental.pallas import tpu_sc as plsc

_NUM_PROMPTS = 12
_PROMPT_LEN = 20
_HIDDEN = 4096
_BATCH = 1024

_NC = 2
_NS = 16
_NW = _NC * _NS

_BPW = _BATCH // _NW
_PIECES = ((0, 8), (8, 8), (16, 4))


def _sc_gather(ridx, table2d):
    mesh = plsc.VectorSubcoreMesh(core_axis_name="c", subcore_axis_name="s")

    @functools.partial(
        pl.kernel,
        mesh=mesh,
        compiler_params=pltpu.CompilerParams(use_tc_tiling_on_sc=True),
        out_type=jax.ShapeDtypeStruct((_PROMPT_LEN, _BATCH, _HIDDEN),
                                      jnp.float32),
        scratch_types=(
            [pltpu.VMEM((_BPW, _PROMPT_LEN), jnp.int32)]
            + [pltpu.VMEM((ln, _HIDDEN), jnp.float32) for _, ln in _PIECES]
            + [pltpu.SemaphoreType.DMA] * 6
        ),
    )
    def k(ridx_hbm, table_hbm, out_hbm, ridx_v, buf0, buf1, buf2,
          g0, g1, g2, s0, s1, s2):
        bufs = (buf0, buf1, buf2)
        gsems = (g0, g1, g2)
        ssems = (s0, s1, s2)
        wid = lax.axis_index("s") * _NC + lax.axis_index("c")
        base = wid * _BPW

        pltpu.sync_copy(ridx_hbm.at[wid], ridx_v)

        def body(j, carry):
            bb = base + j
            for s, (so, ln) in enumerate(_PIECES):
                @pl.when(j > 0)
                def _wait_prev_store():
                    pltpu.make_async_copy(
                        bufs[s],
                        out_hbm.at[pl.ds(so, ln), bb - 1, :],
                        ssems[s],
                    ).wait()

                pltpu.make_async_copy(
                    table_hbm.at[ridx_v.at[j, pl.ds(so, ln)]],
                    bufs[s],
                    gsems[s],
                ).start()
            for s, (so, ln) in enumerate(_PIECES):
                pltpu.make_async_copy(
                    table_hbm.at[ridx_v.at[j, pl.ds(so, ln)]],
                    bufs[s],
                    gsems[s],
                ).wait()
                pltpu.make_async_copy(
                    bufs[s],
                    out_hbm.at[pl.ds(so, ln), bb, :],
                    ssems[s],
                ).start()
            return carry

        lax.fori_loop(0, _BPW, body, 0)

        for s, (so, ln) in enumerate(_PIECES):
            pltpu.make_async_copy(
                bufs[s],
                out_hbm.at[pl.ds(so, ln), base + _BPW - 1, :],
                ssems[s],
            ).wait()

    return k(ridx, table2d)


def kernel(cultural_context, cultural_prompts):
    idx = cultural_context.astype(jnp.int32)
    ridx = (idx.reshape(_NW, _BPW, 1) * _PROMPT_LEN
            + jnp.arange(_PROMPT_LEN, dtype=jnp.int32)[None, None, :])
    table2d = cultural_prompts.reshape(_NUM_PROMPTS * _PROMPT_LEN, _HIDDEN)
    out = _sc_gather(ridx, table2d)
    return jnp.transpose(out, (1, 0, 2))


# speedup vs baseline: 2.0294x; 1.0052x over previous
import functools

import jax
import jax.numpy as jnp
from jax import lax
from jax.experimental import pallas as pl
from jax.experimental.pallas import tpu as pltpu
from jax.experimental.pallas import tpu_sc as plsc

_NUM_PROMPTS = 12
_PROMPT_LEN = 20
_HIDDEN = 4096
_BATCH = 1024

_NC = 2
_NS = 16
_NW = _NC * _NS

_BPW = _BATCH // _NW
_PIECES = ((0, 12), (12, 8))


def _sc_gather(ridx, table2d):
    mesh = plsc.VectorSubcoreMesh(core_axis_name="c", subcore_axis_name="s")

    @functools.partial(
        pl.kernel,
        mesh=mesh,
        compiler_params=pltpu.CompilerParams(use_tc_tiling_on_sc=True),
        out_type=jax.ShapeDtypeStruct((_PROMPT_LEN, _BATCH, _HIDDEN),
                                      jnp.float32),
        scratch_types=(
            [pltpu.VMEM((_BPW, _PROMPT_LEN), jnp.int32)]
            + [pltpu.VMEM((ln, _HIDDEN), jnp.float32) for _, ln in _PIECES]
            + [pltpu.SemaphoreType.DMA] * 4
        ),
    )
    def k(ridx_hbm, table_hbm, out_hbm, ridx_v, buf0, buf1,
          g0, g1, s0, s1):
        bufs = (buf0, buf1)
        gsems = (g0, g1)
        ssems = (s0, s1)
        wid = lax.axis_index("s") * _NC + lax.axis_index("c")
        base = wid * _BPW

        pltpu.sync_copy(ridx_hbm.at[wid], ridx_v)

        def body(j, carry):
            bb = base + j
            for s, (so, ln) in enumerate(_PIECES):
                @pl.when(j > 0)
                def _wait_prev_store():
                    pltpu.make_async_copy(
                        bufs[s],
                        out_hbm.at[pl.ds(so, ln), bb - 1, :],
                        ssems[s],
                    ).wait()

                pltpu.make_async_copy(
                    table_hbm.at[ridx_v.at[j, pl.ds(so, ln)]],
                    bufs[s],
                    gsems[s],
                ).start()
            for s, (so, ln) in enumerate(_PIECES):
                pltpu.make_async_copy(
                    table_hbm.at[ridx_v.at[j, pl.ds(so, ln)]],
                    bufs[s],
                    gsems[s],
                ).wait()
                pltpu.make_async_copy(
                    bufs[s],
                    out_hbm.at[pl.ds(so, ln), bb, :],
                    ssems[s],
                ).start()
            return carry

        lax.fori_loop(0, _BPW, body, 0)

        for s, (so, ln) in enumerate(_PIECES):
            pltpu.make_async_copy(
                bufs[s],
                out_hbm.at[pl.ds(so, ln), base + _BPW - 1, :],
                ssems[s],
            ).wait()

    return k(ridx, table2d)


def kernel(cultural_context, cultural_prompts):
    idx = cultural_context.astype(jnp.int32)
    ridx = (idx.reshape(_NW, _BPW, 1) * _PROMPT_LEN
            + jnp.arange(_PROMPT_LEN, dtype=jnp.int32)[None, None, :])
    table2d = cultural_prompts.reshape(_NUM_PROMPTS * _PROMPT_LEN, _HIDDEN)
    out = _sc_gather(ridx, table2d)
    return jnp.transpose(out, (1, 0, 2))
